# baseline (speedup 1.0000x reference)
import functools
import math

import jax
import jax.numpy as jnp
from jax import lax
from jax.experimental import pallas as pl
from jax.experimental.pallas import tpu as pltpu
from jax.experimental.pallas import tpu_sc as plsc

_LEVELS = 8
_BASE = 8
_FINEST = 160
_LOG2_T = 19
_T = 1 << _LOG2_T
_F = 2
_SCALE = math.exp(math.log(_FINEST / _BASE) / (_LEVELS - 1))
_RES = [int(math.floor(_BASE * (_SCALE ** l))) for l in range(_LEVELS)]
_P2 = 2654435761 - (1 << 32)
_P3 = 805459861
_MASK = _T - 1

_NC = 2
_NS = 16
_NW = _NC * _NS
_CHUNK = 128
_ROWS = _LEVELS * 8 * _CHUNK


def _encode(xf, n, tabf):
    ppw = n // _NW
    nchunk = ppw // _CHUNK

    mesh = plsc.VectorSubcoreMesh(
        core_axis_name="c", subcore_axis_name="s",
        num_cores=_NC, num_subcores=_NS)

    @functools.partial(
        pl.kernel,
        out_type=jax.ShapeDtypeStruct((_LEVELS * _F, n), jnp.float32),
        mesh=mesh,
        scratch_types=[
            pltpu.VMEM((ppw * 3,), jnp.float32),
            pltpu.VMEM((_ROWS,), jnp.int32),
            pltpu.VMEM((_ROWS,), jnp.int32),
            pltpu.VMEM((_ROWS,), jnp.int32),
            pltpu.VMEM((_ROWS,), jnp.int32),
            pltpu.VMEM((_ROWS,), jnp.float32),
            pltpu.VMEM((_ROWS,), jnp.float32),
            pltpu.VMEM((_ROWS,), jnp.float32),
            pltpu.VMEM((_ROWS,), jnp.float32),
            pltpu.VMEM((_LEVELS * _F, _CHUNK), jnp.float32),
            pltpu.VMEM((_LEVELS * _F, _CHUNK), jnp.float32),
            pltpu.SemaphoreType.DMA,
            pltpu.SemaphoreType.DMA,
            pltpu.SemaphoreType.DMA,
            pltpu.SemaphoreType.DMA,
        ],
        compiler_params=pltpu.CompilerParams(needs_layout_passes=False),
    )
    def enc_kernel(x_hbm, tab_hbm, enc_hbm,
                   xs_all, idx0A, idx1A, idx0B, idx1B,
                   vals0A, vals1A, vals0B, vals1B, encA, encB,
                   semGA, semGB, semSA, semSB):
        wid = lax.axis_index("s") * _NC + lax.axis_index("c")
        base0 = wid * ppw
        pltpu.sync_copy(x_hbm.at[pl.ds(base0 * 3, ppw * 3)], xs_all)
        iota = lax.iota(jnp.int32, 16)
        iota3 = iota * 3

        def load_xyz(ci, g):
            b = (ci * _CHUNK + g * 16) * 3 + iota3
            xv = plsc.load_gather(xs_all, [b])
            yv = plsc.load_gather(xs_all, [b + 1])
            zv = plsc.load_gather(xs_all, [b + 2])
            return xv, yv, zv

        def compute_idx(ci, idx0_ref, idx1_ref):
            def grp(g, carry):
                xv, yv, zv = load_xyz(ci, g)
                for l in range(_LEVELS):
                    res = jnp.float32(_RES[l])
                    pxi = (xv * res).astype(jnp.int32)
                    pyi = (yv * res).astype(jnp.int32)
                    pzi = (zv * res).astype(jnp.int32)
                    a = (pxi, pxi + 1)
                    b0 = pyi * jnp.int32(_P2)
                    c0 = pzi * jnp.int32(_P3)
                    bc = (b0 ^ c0, b0 ^ (c0 + jnp.int32(_P3)),
                          (b0 + jnp.int32(_P2)) ^ c0,
                          (b0 + jnp.int32(_P2)) ^ (c0 + jnp.int32(_P3)))
                    for corner in range(8):
                        h = (a[corner >> 2] ^ bc[corner & 3]) & jnp.int32(_MASK)
                        e0 = (h + h) + jnp.int32(2 * l * _T)
                        s = pl.ds((l * 8 + corner) * _CHUNK + g * 16, 16)
                        idx0_ref[s] = e0
                        idx1_ref[s] = e0 + 1
                return carry
            lax.fori_loop(0, _CHUNK // 16, grp, 0)

        def fire(ci, idx0_ref, idx1_ref, v0_ref, v1_ref, sem):
            compute_idx(ci, idx0_ref, idx1_ref)
            pltpu.async_copy(tab_hbm.at[idx0_ref], v0_ref, sem)
            pltpu.async_copy(tab_hbm.at[idx1_ref], v1_ref, sem)

        def drain_gather(idx0_ref, idx1_ref, v0_ref, v1_ref, sem):
            pltpu.make_async_copy(tab_hbm.at[idx0_ref], v0_ref, sem).wait()
            pltpu.make_async_copy(tab_hbm.at[idx1_ref], v1_ref, sem).wait()

        def accum_store(ci, it, v0_ref, v1_ref, enc_v, semS):
            @pl.when(it >= 1)
            def _():
                pltpu.make_async_copy(
                    enc_v, enc_hbm.at[:, pl.ds(base0, _CHUNK)], semS).wait()

            def grp(g, carry):
                xv, yv, zv = load_xyz(ci, g)
                s = pl.ds(g * 16, 16)
                for l in range(_LEVELS):
                    res = jnp.float32(_RES[l])
                    posx = xv * res
                    posy = yv * res
                    posz = zv * res
                    fx = posx - posx.astype(jnp.int32).astype(jnp.float32)
                    fy = posy - posy.astype(jnp.int32).astype(jnp.float32)
                    fz = posz - posz.astype(jnp.int32).astype(jnp.float32)
                    wx = (jnp.float32(1.0) - fx, fx)
                    wy = (jnp.float32(1.0) - fy, fy)
                    wz = (jnp.float32(1.0) - fz, fz)
                    wyz = (wy[0] * wz[0], wy[0] * wz[1],
                           wy[1] * wz[0], wy[1] * wz[1])
                    acc0 = acc1 = None
                    for corner in range(8):
                        sv = pl.ds((l * 8 + corner) * _CHUNK + g * 16, 16)
                        v0 = v0_ref[sv]
                        v1 = v1_ref[sv]
                        wc = wx[corner >> 2] * wyz[corner & 3]
                        if corner == 0:
                            acc0, acc1 = wc * v0, wc * v1
                        else:
                            acc0, acc1 = acc0 + wc * v0, acc1 + wc * v1
                    enc_v[2 * l, s] = acc0
                    enc_v[2 * l + 1, s] = acc1
                return carry
            lax.fori_loop(0, _CHUNK // 16, grp, 0)
            base = base0 + ci * _CHUNK
            pltpu.async_copy(enc_v, enc_hbm.at[:, pl.ds(base, _CHUNK)], semS)

        fire(0, idx0A, idx1A, vals0A, vals1A, semGA)

        def outer(it, carry):
            ci = 2 * it
            fire(ci + 1, idx0B, idx1B, vals0B, vals1B, semGB)
            drain_gather(idx0A, idx1A, vals0A, vals1A, semGA)
            accum_store(ci, it, vals0A, vals1A, encA, semSA)

            @pl.when(it < nchunk // 2 - 1)
            def _():
                fire(ci + 2, idx0A, idx1A, vals0A, vals1A, semGA)

            drain_gather(idx0B, idx1B, vals0B, vals1B, semGB)
            accum_store(ci + 1, it, vals0B, vals1B, encB, semSB)
            return carry

        lax.fori_loop(0, nchunk // 2, outer, 0)
        pltpu.make_async_copy(
            encA, enc_hbm.at[:, pl.ds(base0, _CHUNK)], semSA).wait()
        pltpu.make_async_copy(
            encB, enc_hbm.at[:, pl.ds(base0, _CHUNK)], semSB).wait()

    return enc_kernel(xf, tabf)


_BMLP = 4096


def _mlp_body(enc_ref, w1_ref, w2_ref, w3_ref, out_ref):
    e = enc_ref[...]
    h = jnp.maximum(
        jnp.dot(w1_ref[...], e, preferred_element_type=jnp.float32), 0.0)
    h = jnp.maximum(
        jnp.dot(w2_ref[...], h, preferred_element_type=jnp.float32), 0.0)
    out_ref[...] = jnp.dot(
        w3_ref[...], h, preferred_element_type=jnp.float32)[0]


def _mlp(enc, W1T, W2T, W3T):
    n = enc.shape[1]
    grid = (n // _BMLP,)
    return pl.pallas_call(
        _mlp_body,
        grid=grid,
        in_specs=[
            pl.BlockSpec((_LEVELS * _F, _BMLP), lambda i: (0, i)),
            pl.BlockSpec((64, _LEVELS * _F), lambda i: (0, 0)),
            pl.BlockSpec((64, 64), lambda i: (0, 0)),
            pl.BlockSpec((1, 64), lambda i: (0, 0)),
        ],
        out_specs=pl.BlockSpec((_BMLP,), lambda i: (i,)),
        out_shape=jax.ShapeDtypeStruct((n,), jnp.float32),
    )(enc, W1T, W2T, W3T)


def kernel(x, tables, W1, W2, W3):
    tabf = tables.reshape(-1)
    enc = _encode(x.reshape(-1), x.shape[0], tabf)
    out = _mlp(enc, W1.T, W2.T, W3.T)
    return out.reshape(x.shape[:-1])


# speedup vs baseline: 3.1348x; 3.1348x over previous
import functools
import math

import jax
import jax.numpy as jnp
from jax import lax
from jax.experimental import pallas as pl
from jax.experimental.pallas import tpu as pltpu
from jax.experimental.pallas import tpu_sc as plsc

_LEVELS = 8
_BASE = 8
_FINEST = 160
_LOG2_T = 19
_T = 1 << _LOG2_T
_F = 2
_SCALE = math.exp(math.log(_FINEST / _BASE) / (_LEVELS - 1))
_RES = [int(math.floor(_BASE * (_SCALE ** l))) for l in range(_LEVELS)]
_P2 = 2654435761 - (1 << 32)
_P3 = 805459861
_MASK = _T - 1

_NC = 2
_NS = 16
_NW = _NC * _NS
_CHUNK = 128
_ROWS = _LEVELS * 8 * _CHUNK


def _encode(xc0, xc1, xc2, tab0, tab1, n):
    ppw = n // _NW
    nchunk = ppw // _CHUNK

    mesh = plsc.VectorSubcoreMesh(
        core_axis_name="c", subcore_axis_name="s",
        num_cores=_NC, num_subcores=_NS)

    @functools.partial(
        pl.kernel,
        out_type=jax.ShapeDtypeStruct((_LEVELS * _F, n), jnp.float32),
        mesh=mesh,
        scratch_types=[
            pltpu.VMEM((ppw,), jnp.float32),
            pltpu.VMEM((ppw,), jnp.float32),
            pltpu.VMEM((ppw,), jnp.float32),
            pltpu.VMEM((_ROWS,), jnp.int32),
            pltpu.VMEM((_ROWS,), jnp.int32),
            pltpu.VMEM((_ROWS,), jnp.float32),
            pltpu.VMEM((_ROWS,), jnp.float32),
            pltpu.VMEM((_ROWS,), jnp.float32),
            pltpu.VMEM((_ROWS,), jnp.float32),
            pltpu.VMEM((_LEVELS * _F, _CHUNK), jnp.float32),
            pltpu.VMEM((_LEVELS * _F, _CHUNK), jnp.float32),
            pltpu.SemaphoreType.DMA,
            pltpu.SemaphoreType.DMA,
            pltpu.SemaphoreType.DMA,
            pltpu.SemaphoreType.DMA,
        ],
        compiler_params=pltpu.CompilerParams(needs_layout_passes=False),
    )
    def enc_kernel(x0_hbm, x1_hbm, x2_hbm, tab0_hbm, tab1_hbm, enc_hbm,
                   xs0, xs1, xs2, idxA, idxB,
                   vals0A, vals1A, vals0B, vals1B, encA, encB,
                   semGA, semGB, semSA, semSB):
        wid = lax.axis_index("s") * _NC + lax.axis_index("c")
        base0 = wid * ppw
        pltpu.sync_copy(x0_hbm.at[pl.ds(base0, ppw)], xs0)
        pltpu.sync_copy(x1_hbm.at[pl.ds(base0, ppw)], xs1)
        pltpu.sync_copy(x2_hbm.at[pl.ds(base0, ppw)], xs2)

        def load_xyz(ci, g):
            s = pl.ds(ci * _CHUNK + g * 16, 16)
            return xs0[s], xs1[s], xs2[s]

        def compute_idx(ci, idx_ref):
            def grp(g, carry):
                xv, yv, zv = load_xyz(ci, g)
                for l in range(_LEVELS):
                    res = jnp.float32(_RES[l])
                    pxi = (xv * res).astype(jnp.int32)
                    pyi = (yv * res).astype(jnp.int32)
                    pzi = (zv * res).astype(jnp.int32)
                    a = (pxi, pxi + 1)
                    b0 = pyi * jnp.int32(_P2)
                    c0 = pzi * jnp.int32(_P3)
                    bc = (b0 ^ c0, b0 ^ (c0 + jnp.int32(_P3)),
                          (b0 + jnp.int32(_P2)) ^ c0,
                          (b0 + jnp.int32(_P2)) ^ (c0 + jnp.int32(_P3)))
                    for corner in range(8):
                        h = (a[corner >> 2] ^ bc[corner & 3]) & jnp.int32(_MASK)
                        s = pl.ds((l * 8 + corner) * _CHUNK + g * 16, 16)
                        idx_ref[s] = h + jnp.int32(l * _T)
                return carry
            lax.fori_loop(0, _CHUNK // 16, grp, 0)

        def fire(ci, idx_ref, v0_ref, v1_ref, sem):
            compute_idx(ci, idx_ref)
            pltpu.async_copy(tab0_hbm.at[idx_ref], v0_ref, sem)
            pltpu.async_copy(tab1_hbm.at[idx_ref], v1_ref, sem)

        def drain_gather(idx_ref, v0_ref, v1_ref, sem):
            pltpu.make_async_copy(tab0_hbm.at[idx_ref], v0_ref, sem).wait()
            pltpu.make_async_copy(tab1_hbm.at[idx_ref], v1_ref, sem).wait()

        def accum_store(ci, it, v0_ref, v1_ref, enc_v, semS):
            @pl.when(it >= 1)
            def _():
                pltpu.make_async_copy(
                    enc_v, enc_hbm.at[:, pl.ds(base0, _CHUNK)], semS).wait()

            def grp(g, carry):
                xv, yv, zv = load_xyz(ci, g)
                s = pl.ds(g * 16, 16)
                for l in range(_LEVELS):
                    res = jnp.float32(_RES[l])
                    posx = xv * res
                    posy = yv * res
                    posz = zv * res
                    fx = posx - posx.astype(jnp.int32).astype(jnp.float32)
                    fy = posy - posy.astype(jnp.int32).astype(jnp.float32)
                    fz = posz - posz.astype(jnp.int32).astype(jnp.float32)
                    wx = (jnp.float32(1.0) - fx, fx)
                    wy = (jnp.float32(1.0) - fy, fy)
                    wz = (jnp.float32(1.0) - fz, fz)
                    wyz = (wy[0] * wz[0], wy[0] * wz[1],
                           wy[1] * wz[0], wy[1] * wz[1])
                    acc0 = acc1 = None
                    for corner in range(8):
                        sv = pl.ds((l * 8 + corner) * _CHUNK + g * 16, 16)
                        v0 = v0_ref[sv]
                        v1 = v1_ref[sv]
                        wc = wx[corner >> 2] * wyz[corner & 3]
                        if corner == 0:
                            acc0, acc1 = wc * v0, wc * v1
                        else:
                            acc0, acc1 = acc0 + wc * v0, acc1 + wc * v1
                    enc_v[2 * l, s] = acc0
                    enc_v[2 * l + 1, s] = acc1
                return carry
            lax.fori_loop(0, _CHUNK // 16, grp, 0)
            base = base0 + ci * _CHUNK
            pltpu.async_copy(enc_v, enc_hbm.at[:, pl.ds(base, _CHUNK)], semS)

        fire(0, idxA, vals0A, vals1A, semGA)

        def outer(it, carry):
            ci = 2 * it
            fire(ci + 1, idxB, vals0B, vals1B, semGB)
            drain_gather(idxA, vals0A, vals1A, semGA)
            accum_store(ci, it, vals0A, vals1A, encA, semSA)

            @pl.when(it < nchunk // 2 - 1)
            def _():
                fire(ci + 2, idxA, vals0A, vals1A, semGA)

            drain_gather(idxB, vals0B, vals1B, semGB)
            accum_store(ci + 1, it, vals0B, vals1B, encB, semSB)
            return carry

        lax.fori_loop(0, nchunk // 2, outer, 0)
        pltpu.make_async_copy(
            encA, enc_hbm.at[:, pl.ds(base0, _CHUNK)], semSA).wait()
        pltpu.make_async_copy(
            encB, enc_hbm.at[:, pl.ds(base0, _CHUNK)], semSB).wait()

    return enc_kernel(xc0, xc1, xc2, tab0, tab1)


_BMLP = 4096


def _mlp_body(enc_ref, w1_ref, w2_ref, w3_ref, out_ref):
    e = enc_ref[...]
    h = jnp.maximum(
        jnp.dot(w1_ref[...], e, preferred_element_type=jnp.float32), 0.0)
    h = jnp.maximum(
        jnp.dot(w2_ref[...], h, preferred_element_type=jnp.float32), 0.0)
    out_ref[...] = jnp.dot(
        w3_ref[...], h, preferred_element_type=jnp.float32)[0]


def _mlp(enc, W1T, W2T, W3T):
    n = enc.shape[1]
    grid = (n // _BMLP,)
    return pl.pallas_call(
        _mlp_body,
        grid=grid,
        in_specs=[
            pl.BlockSpec((_LEVELS * _F, _BMLP), lambda i: (0, i)),
            pl.BlockSpec((64, _LEVELS * _F), lambda i: (0, 0)),
            pl.BlockSpec((64, 64), lambda i: (0, 0)),
            pl.BlockSpec((1, 64), lambda i: (0, 0)),
        ],
        out_specs=pl.BlockSpec((_BMLP,), lambda i: (i,)),
        out_shape=jax.ShapeDtypeStruct((n,), jnp.float32),
    )(enc, W1T, W2T, W3T)


def kernel(x, tables, W1, W2, W3):
    n = x.shape[0]
    xc0, xc1, xc2 = x[:, 0], x[:, 1], x[:, 2]
    tab0 = tables[:, :, 0].reshape(-1)
    tab1 = tables[:, :, 1].reshape(-1)
    enc = _encode(xc0, xc1, xc2, tab0, tab1, n)
    out = _mlp(enc, W1.T, W2.T, W3.T)
    return out.reshape(x.shape[:-1])


# speedup vs baseline: 5.4181x; 1.7284x over previous
import functools
import math

import jax
import jax.numpy as jnp
import numpy as np
from jax import lax
from jax.experimental import pallas as pl
from jax.experimental.pallas import tpu as pltpu
from jax.experimental.pallas import tpu_sc as plsc

_LEVELS = 8
_BASE = 8
_FINEST = 160
_LOG2_T = 19
_T = 1 << _LOG2_T
_F = 2
_SCALE = math.exp(math.log(_FINEST / _BASE) / (_LEVELS - 1))
_RES = [int(math.floor(_BASE * (_SCALE ** l))) for l in range(_LEVELS)]
_P2 = 2654435761 - (1 << 32)
_P3 = 805459861
_MASK = _T - 1

_NC = 2
_NS = 16
_NW = _NC * _NS
_CHUNK = 128
_NLOC = 4
_NSTR = _LEVELS - _NLOC
_ROWS = _NSTR * 8 * _CHUNK

_S = [_RES[l] + 1 for l in range(_NLOC)]
_GOFF = [0]
for _l in range(_NLOC):
    _GOFF.append(_GOFF[-1] + _S[_l] ** 3)
_GSIZE = _GOFF[_NLOC]
_GWAVE = 1024
_GPAD = ((_GSIZE + _GWAVE - 1) // _GWAVE) * _GWAVE


def _grid_hash_indices() -> np.ndarray:
    out = np.zeros((_GPAD,), dtype=np.int32)
    pos = 0
    p2 = np.uint32(2654435761)
    p3 = np.uint32(_P3)
    for l in range(_NLOC):
        s = _S[l]
        ii, jj, kk = np.meshgrid(
            np.arange(s, dtype=np.uint32),
            np.arange(s, dtype=np.uint32),
            np.arange(s, dtype=np.uint32), indexing="ij")
        h = (ii ^ (jj * p2) ^ (kk * p3)) & np.uint32(_MASK)
        out[pos:pos + s ** 3] = (h + np.uint32(l * _T)).astype(np.int32).ravel()
        pos += s ** 3
    return out


def _encode(xc0, xc1, xc2, tab0, tab1, ghidx, n):
    ppw = n // _NW
    nchunk = ppw // _CHUNK

    mesh = plsc.VectorSubcoreMesh(
        core_axis_name="c", subcore_axis_name="s",
        num_cores=_NC, num_subcores=_NS)

    @functools.partial(
        pl.kernel,
        out_type=jax.ShapeDtypeStruct((_LEVELS * _F, n), jnp.float32),
        mesh=mesh,
        scratch_types=[
            pltpu.VMEM((_GPAD,), jnp.float32),
            pltpu.VMEM((_GPAD,), jnp.float32),
            pltpu.VMEM((_GWAVE,), jnp.int32),
            pltpu.VMEM((_CHUNK,), jnp.float32),
            pltpu.VMEM((_CHUNK,), jnp.float32),
            pltpu.VMEM((_CHUNK,), jnp.float32),
            pltpu.VMEM((_CHUNK,), jnp.float32),
            pltpu.VMEM((_CHUNK,), jnp.float32),
            pltpu.VMEM((_CHUNK,), jnp.float32),
            pltpu.VMEM((_ROWS,), jnp.int32),
            pltpu.VMEM((_ROWS,), jnp.int32),
            pltpu.VMEM((_ROWS,), jnp.float32),
            pltpu.VMEM((_ROWS,), jnp.float32),
            pltpu.VMEM((_ROWS,), jnp.float32),
            pltpu.VMEM((_ROWS,), jnp.float32),
            pltpu.VMEM((_LEVELS * _F, _CHUNK), jnp.float32),
            pltpu.VMEM((_LEVELS * _F, _CHUNK), jnp.float32),
            pltpu.SemaphoreType.DMA,
            pltpu.SemaphoreType.DMA,
            pltpu.SemaphoreType.DMA,
            pltpu.SemaphoreType.DMA,
            pltpu.SemaphoreType.DMA,
            pltpu.SemaphoreType.DMA,
        ],
        compiler_params=pltpu.CompilerParams(needs_layout_passes=False),
    )
    def enc_kernel(x0_hbm, x1_hbm, x2_hbm, tab0_hbm, tab1_hbm, gh_hbm,
                   enc_hbm,
                   grid0, grid1, ghv,
                   xsA0, xsA1, xsA2, xsB0, xsB1, xsB2,
                   idxA, idxB, vals0A, vals1A, vals0B, vals1B, encA, encB,
                   semGA, semGB, semSA, semSB, semXA, semXB):
        wid = lax.axis_index("s") * _NC + lax.axis_index("c")
        base0 = wid * ppw

        def build_wave(w, carry):
            off = w * _GWAVE
            pltpu.sync_copy(gh_hbm.at[pl.ds(off, _GWAVE)], ghv)
            pltpu.async_copy(
                tab0_hbm.at[ghv], grid0.at[pl.ds(off, _GWAVE)], semGA)
            pltpu.async_copy(
                tab1_hbm.at[ghv], grid1.at[pl.ds(off, _GWAVE)], semGA)
            pltpu.make_async_copy(
                tab0_hbm.at[ghv], grid0.at[pl.ds(off, _GWAVE)], semGA).wait()
            pltpu.make_async_copy(
                tab1_hbm.at[ghv], grid1.at[pl.ds(off, _GWAVE)], semGA).wait()
            return carry
        lax.fori_loop(0, _GPAD // _GWAVE, build_wave, 0)

        def xload(ci, b0, b1, b2, semX):
            base = base0 + ci * _CHUNK
            pltpu.async_copy(x0_hbm.at[pl.ds(base, _CHUNK)], b0, semX)
            pltpu.async_copy(x1_hbm.at[pl.ds(base, _CHUNK)], b1, semX)
            pltpu.async_copy(x2_hbm.at[pl.ds(base, _CHUNK)], b2, semX)

        def xdrain(ci, b0, b1, b2, semX):
            base = base0 + ci * _CHUNK
            pltpu.make_async_copy(x0_hbm.at[pl.ds(base, _CHUNK)], b0, semX).wait()
            pltpu.make_async_copy(x1_hbm.at[pl.ds(base, _CHUNK)], b1, semX).wait()
            pltpu.make_async_copy(x2_hbm.at[pl.ds(base, _CHUNK)], b2, semX).wait()

        def compute_idx(xs, idx_ref):
            b0s, b1s, b2s = xs

            def grp(g, carry):
                s = pl.ds(g * 16, 16)
                xv, yv, zv = b0s[s], b1s[s], b2s[s]
                for li, l in enumerate(range(_NLOC, _LEVELS)):
                    res = jnp.float32(_RES[l])
                    pxi = (xv * res).astype(jnp.int32)
                    pyi = (yv * res).astype(jnp.int32)
                    pzi = (zv * res).astype(jnp.int32)
                    a = (pxi, pxi + 1)
                    b0 = pyi * jnp.int32(_P2)
                    c0 = pzi * jnp.int32(_P3)
                    bc = (b0 ^ c0, b0 ^ (c0 + jnp.int32(_P3)),
                          (b0 + jnp.int32(_P2)) ^ c0,
                          (b0 + jnp.int32(_P2)) ^ (c0 + jnp.int32(_P3)))
                    for corner in range(8):
                        h = (a[corner >> 2] ^ bc[corner & 3]) & jnp.int32(_MASK)
                        sl = pl.ds((li * 8 + corner) * _CHUNK + g * 16, 16)
                        idx_ref[sl] = h + jnp.int32(l * _T)
                return carry
            lax.fori_loop(0, _CHUNK // 16, grp, 0)

        def fire(ci, xs, semX, idx_ref, v0_ref, v1_ref, sem):
            xdrain(ci, *xs, semX)
            compute_idx(xs, idx_ref)
            pltpu.async_copy(tab0_hbm.at[idx_ref], v0_ref, sem)
            pltpu.async_copy(tab1_hbm.at[idx_ref], v1_ref, sem)

        def drain_gather(idx_ref, v0_ref, v1_ref, sem):
            pltpu.make_async_copy(tab0_hbm.at[idx_ref], v0_ref, sem).wait()
            pltpu.make_async_copy(tab1_hbm.at[idx_ref], v1_ref, sem).wait()

        def accum_store(ci, it, xs, semX, v0_ref, v1_ref, enc_v, semS):
            b0s, b1s, b2s = xs

            @pl.when(it >= 1)
            def _():
                pltpu.make_async_copy(
                    enc_v, enc_hbm.at[:, pl.ds(base0, _CHUNK)], semS).wait()

            def grp(g, carry):
                s = pl.ds(g * 16, 16)
                xv, yv, zv = b0s[s], b1s[s], b2s[s]
                for l in range(_NLOC):
                    res = jnp.float32(_RES[l])
                    posx = xv * res
                    posy = yv * res
                    posz = zv * res
                    pxi = posx.astype(jnp.int32)
                    pyi = posy.astype(jnp.int32)
                    pzi = posz.astype(jnp.int32)
                    fx = posx - pxi.astype(jnp.float32)
                    fy = posy - pyi.astype(jnp.float32)
                    fz = posz - pzi.astype(jnp.float32)
                    wx = (jnp.float32(1.0) - fx, fx)
                    wy = (jnp.float32(1.0) - fy, fy)
                    wz = (jnp.float32(1.0) - fz, fz)
                    wyz = (wy[0] * wz[0], wy[0] * wz[1],
                           wy[1] * wz[0], wy[1] * wz[1])
                    S = _S[l]
                    cell = (pxi * S + pyi) * S + pzi
                    acc0 = acc1 = None
                    for corner in range(8):
                        i3, j3, k3 = corner >> 2, (corner >> 1) & 1, corner & 1
                        doff = (i3 * S + j3) * S + k3 + _GOFF[l]
                        cvec = cell + jnp.int32(doff)
                        v0 = plsc.load_gather(grid0, [cvec])
                        v1 = plsc.load_gather(grid1, [cvec])
                        wc = wx[i3] * wyz[corner & 3]
                        if corner == 0:
                            acc0, acc1 = wc * v0, wc * v1
                        else:
                            acc0, acc1 = acc0 + wc * v0, acc1 + wc * v1
                    enc_v[2 * l, s] = acc0
                    enc_v[2 * l + 1, s] = acc1
                for li, l in enumerate(range(_NLOC, _LEVELS)):
                    res = jnp.float32(_RES[l])
                    posx = xv * res
                    posy = yv * res
                    posz = zv * res
                    fx = posx - posx.astype(jnp.int32).astype(jnp.float32)
                    fy = posy - posy.astype(jnp.int32).astype(jnp.float32)
                    fz = posz - posz.astype(jnp.int32).astype(jnp.float32)
                    wx = (jnp.float32(1.0) - fx, fx)
                    wy = (jnp.float32(1.0) - fy, fy)
                    wz = (jnp.float32(1.0) - fz, fz)
                    wyz = (wy[0] * wz[0], wy[0] * wz[1],
                           wy[1] * wz[0], wy[1] * wz[1])
                    acc0 = acc1 = None
                    for corner in range(8):
                        sv = pl.ds((li * 8 + corner) * _CHUNK + g * 16, 16)
                        v0 = v0_ref[sv]
                        v1 = v1_ref[sv]
                        wc = wx[corner >> 2] * wyz[corner & 3]
                        if corner == 0:
                            acc0, acc1 = wc * v0, wc * v1
                        else:
                            acc0, acc1 = acc0 + wc * v0, acc1 + wc * v1
                    enc_v[2 * l, s] = acc0
                    enc_v[2 * l + 1, s] = acc1
                return carry
            lax.fori_loop(0, _CHUNK // 16, grp, 0)
            base = base0 + ci * _CHUNK
            pltpu.async_copy(enc_v, enc_hbm.at[:, pl.ds(base, _CHUNK)], semS)

            @pl.when(ci + 2 < nchunk)
            def _():
                xload(ci + 2, b0s, b1s, b2s, semX)

        xsA = (xsA0, xsA1, xsA2)
        xsB = (xsB0, xsB1, xsB2)
        xload(0, *xsA, semXA)
        xload(1, *xsB, semXB)
        fire(0, xsA, semXA, idxA, vals0A, vals1A, semGA)

        def outer(it, carry):
            ci = 2 * it
            fire(ci + 1, xsB, semXB, idxB, vals0B, vals1B, semGB)
            drain_gather(idxA, vals0A, vals1A, semGA)
            accum_store(ci, it, xsA, semXA, vals0A, vals1A, encA, semSA)

            @pl.when(it < nchunk // 2 - 1)
            def _():
                fire(ci + 2, xsA, semXA, idxA, vals0A, vals1A, semGA)

            drain_gather(idxB, vals0B, vals1B, semGB)
            accum_store(ci + 1, it, xsB, semXB, vals0B, vals1B, encB, semSB)
            return carry

        lax.fori_loop(0, nchunk // 2, outer, 0)
        pltpu.make_async_copy(
            encA, enc_hbm.at[:, pl.ds(base0, _CHUNK)], semSA).wait()
        pltpu.make_async_copy(
            encB, enc_hbm.at[:, pl.ds(base0, _CHUNK)], semSB).wait()

    return enc_kernel(xc0, xc1, xc2, tab0, tab1, ghidx)


_BMLP = 4096


def _mlp_body(enc_ref, w1_ref, w2_ref, w3_ref, out_ref):
    e = enc_ref[...]
    h = jnp.maximum(
        jnp.dot(w1_ref[...], e, preferred_element_type=jnp.float32), 0.0)
    h = jnp.maximum(
        jnp.dot(w2_ref[...], h, preferred_element_type=jnp.float32), 0.0)
    out_ref[...] = jnp.dot(
        w3_ref[...], h, preferred_element_type=jnp.float32)[0]


def _mlp(enc, W1T, W2T, W3T):
    n = enc.shape[1]
    grid = (n // _BMLP,)
    return pl.pallas_call(
        _mlp_body,
        grid=grid,
        in_specs=[
            pl.BlockSpec((_LEVELS * _F, _BMLP), lambda i: (0, i)),
            pl.BlockSpec((64, _LEVELS * _F), lambda i: (0, 0)),
            pl.BlockSpec((64, 64), lambda i: (0, 0)),
            pl.BlockSpec((1, 64), lambda i: (0, 0)),
        ],
        out_specs=pl.BlockSpec((_BMLP,), lambda i: (i,)),
        out_shape=jax.ShapeDtypeStruct((n,), jnp.float32),
    )(enc, W1T, W2T, W3T)


_GHIDX = _grid_hash_indices()


def kernel(x, tables, W1, W2, W3):
    n = x.shape[0]
    xc0, xc1, xc2 = x[:, 0], x[:, 1], x[:, 2]
    tab0 = tables[:, :, 0].reshape(-1)
    tab1 = tables[:, :, 1].reshape(-1)
    ghidx = jnp.asarray(_GHIDX)
    enc = _encode(xc0, xc1, xc2, tab0, tab1, ghidx, n)
    out = _mlp(enc, W1.T, W2.T, W3.T)
    return out.reshape(x.shape[:-1])


# speedup vs baseline: 8.2235x; 1.5178x over previous
import functools
import math

import jax
import jax.numpy as jnp
import numpy as np
from jax import lax
from jax.experimental import pallas as pl
from jax.experimental.pallas import tpu as pltpu
from jax.experimental.pallas import tpu_sc as plsc

_LEVELS = 8
_BASE = 8
_FINEST = 160
_LOG2_T = 19
_T = 1 << _LOG2_T
_F = 2
_SCALE = math.exp(math.log(_FINEST / _BASE) / (_LEVELS - 1))
_RES = [int(math.floor(_BASE * (_SCALE ** l))) for l in range(_LEVELS)]
_P2 = 2654435761 - (1 << 32)
_P3 = 805459861
_MASK = _T - 1

_NC = 2
_NS = 16
_NW = _NC * _NS
_CHUNK = 128
_NLOC = 4
_NSTR = _LEVELS - _NLOC
_ROWS = _NSTR * 8 * _CHUNK

_S = [_RES[l] + 1 for l in range(_NLOC)]
_GOFF = [0]
for _l in range(_NLOC):
    _GOFF.append(_GOFF[-1] + _S[_l] ** 3)
_GSIZE = _GOFF[_NLOC]
_GWAVE = 1024
_GPAD = ((_GSIZE + _GWAVE - 1) // _GWAVE) * _GWAVE


def _grid_hash_indices() -> np.ndarray:
    out = np.zeros((_GPAD,), dtype=np.int32)
    pos = 0
    p2 = np.uint32(2654435761)
    p3 = np.uint32(_P3)
    for l in range(_NLOC):
        s = _S[l]
        ii, jj, kk = np.meshgrid(
            np.arange(s, dtype=np.uint32),
            np.arange(s, dtype=np.uint32),
            np.arange(s, dtype=np.uint32), indexing="ij")
        h = (ii ^ (jj * p2) ^ (kk * p3)) & np.uint32(_MASK)
        out[pos:pos + s ** 3] = (h + np.uint32(l * _T)).astype(np.int32).ravel()
        pos += s ** 3
    return out


def _encode(xc0, xc1, xc2, tab0, tab1, tpk, ghidx, n):
    ppw = n // _NW
    nchunk = ppw // _CHUNK

    mesh = plsc.VectorSubcoreMesh(
        core_axis_name="c", subcore_axis_name="s",
        num_cores=_NC, num_subcores=_NS)

    @functools.partial(
        pl.kernel,
        out_type=jax.ShapeDtypeStruct((_LEVELS * _F, n), jnp.float32),
        mesh=mesh,
        scratch_types=[
            pltpu.VMEM((_GPAD,), jnp.float32),
            pltpu.VMEM((_GPAD,), jnp.float32),
            pltpu.VMEM((_GWAVE,), jnp.int32),
            pltpu.VMEM((_CHUNK,), jnp.float32),
            pltpu.VMEM((_CHUNK,), jnp.float32),
            pltpu.VMEM((_CHUNK,), jnp.float32),
            pltpu.VMEM((_CHUNK,), jnp.float32),
            pltpu.VMEM((_CHUNK,), jnp.float32),
            pltpu.VMEM((_CHUNK,), jnp.float32),
            pltpu.VMEM((_ROWS,), jnp.int32),
            pltpu.VMEM((_ROWS,), jnp.int32),
            pltpu.VMEM((_ROWS,), jnp.float32),
            pltpu.VMEM((_ROWS,), jnp.float32),
            pltpu.VMEM((_LEVELS * _F, _CHUNK), jnp.float32),
            pltpu.VMEM((_LEVELS * _F, _CHUNK), jnp.float32),
            pltpu.SemaphoreType.DMA,
            pltpu.SemaphoreType.DMA,
            pltpu.SemaphoreType.DMA,
            pltpu.SemaphoreType.DMA,
            pltpu.SemaphoreType.DMA,
            pltpu.SemaphoreType.DMA,
        ],
        compiler_params=pltpu.CompilerParams(needs_layout_passes=False),
    )
    def enc_kernel(x0_hbm, x1_hbm, x2_hbm, tab0_hbm, tab1_hbm, tpk_hbm,
                   gh_hbm, enc_hbm,
                   grid0, grid1, ghv,
                   xsA0, xsA1, xsA2, xsB0, xsB1, xsB2,
                   idxA, idxB, valsA, valsB, encA, encB,
                   semGA, semGB, semSA, semSB, semXA, semXB):
        wid = lax.axis_index("s") * _NC + lax.axis_index("c")
        base0 = wid * ppw

        def build_wave(w, carry):
            off = w * _GWAVE
            pltpu.sync_copy(gh_hbm.at[pl.ds(off, _GWAVE)], ghv)
            pltpu.async_copy(
                tab0_hbm.at[ghv], grid0.at[pl.ds(off, _GWAVE)], semGA)
            pltpu.async_copy(
                tab1_hbm.at[ghv], grid1.at[pl.ds(off, _GWAVE)], semGA)
            pltpu.make_async_copy(
                tab0_hbm.at[ghv], grid0.at[pl.ds(off, _GWAVE)], semGA).wait()
            pltpu.make_async_copy(
                tab1_hbm.at[ghv], grid1.at[pl.ds(off, _GWAVE)], semGA).wait()
            return carry
        lax.fori_loop(0, _GPAD // _GWAVE, build_wave, 0)

        def xload(ci, b0, b1, b2, semX):
            base = base0 + ci * _CHUNK
            pltpu.async_copy(x0_hbm.at[pl.ds(base, _CHUNK)], b0, semX)
            pltpu.async_copy(x1_hbm.at[pl.ds(base, _CHUNK)], b1, semX)
            pltpu.async_copy(x2_hbm.at[pl.ds(base, _CHUNK)], b2, semX)

        def xdrain(ci, b0, b1, b2, semX):
            base = base0 + ci * _CHUNK
            pltpu.make_async_copy(x0_hbm.at[pl.ds(base, _CHUNK)], b0, semX).wait()
            pltpu.make_async_copy(x1_hbm.at[pl.ds(base, _CHUNK)], b1, semX).wait()
            pltpu.make_async_copy(x2_hbm.at[pl.ds(base, _CHUNK)], b2, semX).wait()

        def compute_idx(xs, idx_ref):
            b0s, b1s, b2s = xs

            def grp(g, carry):
                s = pl.ds(g * 16, 16)
                xv, yv, zv = b0s[s], b1s[s], b2s[s]
                for li, l in enumerate(range(_NLOC, _LEVELS)):
                    res = jnp.float32(_RES[l])
                    pxi = (xv * res).astype(jnp.int32)
                    pyi = (yv * res).astype(jnp.int32)
                    pzi = (zv * res).astype(jnp.int32)
                    a = (pxi, pxi + 1)
                    b0 = pyi * jnp.int32(_P2)
                    c0 = pzi * jnp.int32(_P3)
                    bc = (b0 ^ c0, b0 ^ (c0 + jnp.int32(_P3)),
                          (b0 + jnp.int32(_P2)) ^ c0,
                          (b0 + jnp.int32(_P2)) ^ (c0 + jnp.int32(_P3)))
                    for corner in range(8):
                        h = (a[corner >> 2] ^ bc[corner & 3]) & jnp.int32(_MASK)
                        sl = pl.ds((li * 8 + corner) * _CHUNK + g * 16, 16)
                        idx_ref[sl] = h + jnp.int32(li * _T)
                return carry
            lax.fori_loop(0, _CHUNK // 16, grp, 0)

        def fire(ci, xs, semX, idx_ref, v_ref, sem):
            xdrain(ci, *xs, semX)
            compute_idx(xs, idx_ref)
            pltpu.async_copy(tpk_hbm.at[idx_ref], v_ref, sem)

        def drain_gather(idx_ref, v_ref, sem):
            pltpu.make_async_copy(tpk_hbm.at[idx_ref], v_ref, sem).wait()

        def accum_store(ci, it, xs, semX, v_ref, enc_v, semS):
            b0s, b1s, b2s = xs

            @pl.when(it >= 1)
            def _():
                pltpu.make_async_copy(
                    enc_v, enc_hbm.at[:, pl.ds(base0, _CHUNK)], semS).wait()

            def grp(g, carry):
                s = pl.ds(g * 16, 16)
                xv, yv, zv = b0s[s], b1s[s], b2s[s]
                for l in range(_NLOC):
                    res = jnp.float32(_RES[l])
                    posx = xv * res
                    posy = yv * res
                    posz = zv * res
                    pxi = posx.astype(jnp.int32)
                    pyi = posy.astype(jnp.int32)
                    pzi = posz.astype(jnp.int32)
                    fx = posx - pxi.astype(jnp.float32)
                    fy = posy - pyi.astype(jnp.float32)
                    fz = posz - pzi.astype(jnp.float32)
                    wx = (jnp.float32(1.0) - fx, fx)
                    wy = (jnp.float32(1.0) - fy, fy)
                    wz = (jnp.float32(1.0) - fz, fz)
                    wyz = (wy[0] * wz[0], wy[0] * wz[1],
                           wy[1] * wz[0], wy[1] * wz[1])
                    S = _S[l]
                    cell = (pxi * S + pyi) * S + pzi
                    acc0 = acc1 = None
                    for corner in range(8):
                        i3, j3, k3 = corner >> 2, (corner >> 1) & 1, corner & 1
                        doff = (i3 * S + j3) * S + k3 + _GOFF[l]
                        cvec = cell + jnp.int32(doff)
                        v0 = plsc.load_gather(grid0, [cvec])
                        v1 = plsc.load_gather(grid1, [cvec])
                        wc = wx[i3] * wyz[corner & 3]
                        if corner == 0:
                            acc0, acc1 = wc * v0, wc * v1
                        else:
                            acc0, acc1 = acc0 + wc * v0, acc1 + wc * v1
                    enc_v[2 * l, s] = acc0
                    enc_v[2 * l + 1, s] = acc1
                for li, l in enumerate(range(_NLOC, _LEVELS)):
                    res = jnp.float32(_RES[l])
                    posx = xv * res
                    posy = yv * res
                    posz = zv * res
                    fx = posx - posx.astype(jnp.int32).astype(jnp.float32)
                    fy = posy - posy.astype(jnp.int32).astype(jnp.float32)
                    fz = posz - posz.astype(jnp.int32).astype(jnp.float32)
                    wx = (jnp.float32(1.0) - fx, fx)
                    wy = (jnp.float32(1.0) - fy, fy)
                    wz = (jnp.float32(1.0) - fz, fz)
                    wyz = (wy[0] * wz[0], wy[0] * wz[1],
                           wy[1] * wz[0], wy[1] * wz[1])
                    acc0 = acc1 = None
                    for corner in range(8):
                        sv = pl.ds((li * 8 + corner) * _CHUNK + g * 16, 16)
                        ui = plsc.bitcast(v_ref[sv], jnp.int32)
                        v0 = plsc.bitcast(ui & jnp.int32(-65536), jnp.float32)
                        v1 = plsc.bitcast(
                            lax.shift_left(ui, jnp.int32(16)), jnp.float32)
                        wc = wx[corner >> 2] * wyz[corner & 3]
                        if corner == 0:
                            acc0, acc1 = wc * v0, wc * v1
                        else:
                            acc0, acc1 = acc0 + wc * v0, acc1 + wc * v1
                    enc_v[2 * l, s] = acc0
                    enc_v[2 * l + 1, s] = acc1
                return carry
            lax.fori_loop(0, _CHUNK // 16, grp, 0)
            base = base0 + ci * _CHUNK
            pltpu.async_copy(enc_v, enc_hbm.at[:, pl.ds(base, _CHUNK)], semS)

            @pl.when(ci + 2 < nchunk)
            def _():
                xload(ci + 2, b0s, b1s, b2s, semX)

        xsA = (xsA0, xsA1, xsA2)
        xsB = (xsB0, xsB1, xsB2)
        xload(0, *xsA, semXA)
        xload(1, *xsB, semXB)
        fire(0, xsA, semXA, idxA, valsA, semGA)

        def outer(it, carry):
            ci = 2 * it
            fire(ci + 1, xsB, semXB, idxB, valsB, semGB)
            drain_gather(idxA, valsA, semGA)
            accum_store(ci, it, xsA, semXA, valsA, encA, semSA)

            @pl.when(it < nchunk // 2 - 1)
            def _():
                fire(ci + 2, xsA, semXA, idxA, valsA, semGA)

            drain_gather(idxB, valsB, semGB)
            accum_store(ci + 1, it, xsB, semXB, valsB, encB, semSB)
            return carry

        lax.fori_loop(0, nchunk // 2, outer, 0)
        pltpu.make_async_copy(
            encA, enc_hbm.at[:, pl.ds(base0, _CHUNK)], semSA).wait()
        pltpu.make_async_copy(
            encB, enc_hbm.at[:, pl.ds(base0, _CHUNK)], semSB).wait()

    return enc_kernel(xc0, xc1, xc2, tab0, tab1, tpk, ghidx)


_BMLP = 4096


def _mlp_body(enc_ref, w1_ref, w2_ref, w3_ref, out_ref):
    e = enc_ref[...]
    h = jnp.maximum(
        jnp.dot(w1_ref[...], e, preferred_element_type=jnp.float32), 0.0)
    h = jnp.maximum(
        jnp.dot(w2_ref[...], h, preferred_element_type=jnp.float32), 0.0)
    out_ref[...] = jnp.dot(
        w3_ref[...], h, preferred_element_type=jnp.float32)[0]


def _mlp(enc, W1T, W2T, W3T):
    n = enc.shape[1]
    grid = (n // _BMLP,)
    return pl.pallas_call(
        _mlp_body,
        grid=grid,
        in_specs=[
            pl.BlockSpec((_LEVELS * _F, _BMLP), lambda i: (0, i)),
            pl.BlockSpec((64, _LEVELS * _F), lambda i: (0, 0)),
            pl.BlockSpec((64, 64), lambda i: (0, 0)),
            pl.BlockSpec((1, 64), lambda i: (0, 0)),
        ],
        out_specs=pl.BlockSpec((_BMLP,), lambda i: (i,)),
        out_shape=jax.ShapeDtypeStruct((n,), jnp.float32),
    )(enc, W1T, W2T, W3T)


_GHIDX = _grid_hash_indices()


def kernel(x, tables, W1, W2, W3):
    n = x.shape[0]
    xc0, xc1, xc2 = x[:, 0], x[:, 1], x[:, 2]
    tab0 = tables[:, :, 0].reshape(-1)
    tab1 = tables[:, :, 1].reshape(-1)
    ts = tables[_NLOC:]
    b0 = lax.bitcast_convert_type(
        ts[:, :, 0].astype(jnp.bfloat16), jnp.uint16).astype(jnp.uint32)
    b1 = lax.bitcast_convert_type(
        ts[:, :, 1].astype(jnp.bfloat16), jnp.uint16).astype(jnp.uint32)
    tpk = lax.bitcast_convert_type(
        (b0 << 16) | b1, jnp.float32).reshape(-1)
    ghidx = jnp.asarray(_GHIDX)
    enc = _encode(xc0, xc1, xc2, tab0, tab1, tpk, ghidx, n)
    out = _mlp(enc, W1.T, W2.T, W3.T)
    return out.reshape(x.shape[:-1])


# speedup vs baseline: 9.0388x; 1.0991x over previous
import functools
import math

import jax
import jax.numpy as jnp
import numpy as np
from jax import lax
from jax.experimental import pallas as pl
from jax.experimental.pallas import tpu as pltpu
from jax.experimental.pallas import tpu_sc as plsc

_LEVELS = 8
_BASE = 8
_FINEST = 160
_LOG2_T = 19
_T = 1 << _LOG2_T
_F = 2
_SCALE = math.exp(math.log(_FINEST / _BASE) / (_LEVELS - 1))
_RES = [int(math.floor(_BASE * (_SCALE ** l))) for l in range(_LEVELS)]
_P2 = 2654435761 - (1 << 32)
_P3 = 805459861
_MASK = _T - 1

_NC = 2
_NS = 16
_NW = _NC * _NS
_CHUNK = 128
_NLOC = 4
_NSTR = _LEVELS - _NLOC
_ROWS = _NSTR * 8 * _CHUNK

_S = [_RES[l] + 1 for l in range(_NLOC)]
_GOFF = [0]
for _l in range(_NLOC):
    _GOFF.append(_GOFF[-1] + _S[_l] ** 3)
_GSIZE = _GOFF[_NLOC]
_GWAVE = 1024
_GPAD = ((_GSIZE + _GWAVE - 1) // _GWAVE) * _GWAVE


def _grid_hash_indices() -> np.ndarray:
    out = np.zeros((_GPAD,), dtype=np.int32)
    pos = 0
    p2 = np.uint32(2654435761)
    p3 = np.uint32(_P3)
    for l in range(_NLOC):
        s = _S[l]
        ii, jj, kk = np.meshgrid(
            np.arange(s, dtype=np.uint32),
            np.arange(s, dtype=np.uint32),
            np.arange(s, dtype=np.uint32), indexing="ij")
        h = (ii ^ (jj * p2) ^ (kk * p3)) & np.uint32(_MASK)
        out[pos:pos + s ** 3] = (h + np.uint32(l * _T)).astype(np.int32).ravel()
        pos += s ** 3
    return out


def _encode(xc0, xc1, xc2, tpk, ghidx, n):
    ppw = n // _NW
    nchunk = ppw // _CHUNK

    mesh = plsc.VectorSubcoreMesh(
        core_axis_name="c", subcore_axis_name="s",
        num_cores=_NC, num_subcores=_NS)

    @functools.partial(
        pl.kernel,
        out_type=jax.ShapeDtypeStruct((_LEVELS * _F, n), jnp.float32),
        mesh=mesh,
        scratch_types=[
            pltpu.VMEM((_GPAD,), jnp.float32),
            pltpu.VMEM((_GWAVE,), jnp.int32),
            pltpu.VMEM((_CHUNK,), jnp.float32),
            pltpu.VMEM((_CHUNK,), jnp.float32),
            pltpu.VMEM((_CHUNK,), jnp.float32),
            pltpu.VMEM((_CHUNK,), jnp.float32),
            pltpu.VMEM((_CHUNK,), jnp.float32),
            pltpu.VMEM((_CHUNK,), jnp.float32),
            pltpu.VMEM((_ROWS,), jnp.int32),
            pltpu.VMEM((_ROWS,), jnp.int32),
            pltpu.VMEM((_ROWS,), jnp.float32),
            pltpu.VMEM((_ROWS,), jnp.float32),
            pltpu.VMEM((_LEVELS * _F, _CHUNK), jnp.float32),
            pltpu.VMEM((_LEVELS * _F, _CHUNK), jnp.float32),
            pltpu.SemaphoreType.DMA,
            pltpu.SemaphoreType.DMA,
            pltpu.SemaphoreType.DMA,
            pltpu.SemaphoreType.DMA,
            pltpu.SemaphoreType.DMA,
            pltpu.SemaphoreType.DMA,
        ],
        compiler_params=pltpu.CompilerParams(needs_layout_passes=False),
    )
    def enc_kernel(x0_hbm, x1_hbm, x2_hbm, tpk_hbm,
                   gh_hbm, enc_hbm,
                   gridP, ghv,
                   xsA0, xsA1, xsA2, xsB0, xsB1, xsB2,
                   idxA, idxB, valsA, valsB, encA, encB,
                   semGA, semGB, semSA, semSB, semXA, semXB):
        wid = lax.axis_index("s") * _NC + lax.axis_index("c")
        base0 = wid * ppw

        def build_wave(w, carry):
            off = w * _GWAVE
            pltpu.sync_copy(gh_hbm.at[pl.ds(off, _GWAVE)], ghv)
            pltpu.async_copy(
                tpk_hbm.at[ghv], gridP.at[pl.ds(off, _GWAVE)], semGA)
            pltpu.make_async_copy(
                tpk_hbm.at[ghv], gridP.at[pl.ds(off, _GWAVE)], semGA).wait()
            return carry
        lax.fori_loop(0, _GPAD // _GWAVE, build_wave, 0)

        def xload(ci, b0, b1, b2, semX):
            base = base0 + ci * _CHUNK
            pltpu.async_copy(x0_hbm.at[pl.ds(base, _CHUNK)], b0, semX)
            pltpu.async_copy(x1_hbm.at[pl.ds(base, _CHUNK)], b1, semX)
            pltpu.async_copy(x2_hbm.at[pl.ds(base, _CHUNK)], b2, semX)

        def xdrain(ci, b0, b1, b2, semX):
            base = base0 + ci * _CHUNK
            pltpu.make_async_copy(x0_hbm.at[pl.ds(base, _CHUNK)], b0, semX).wait()
            pltpu.make_async_copy(x1_hbm.at[pl.ds(base, _CHUNK)], b1, semX).wait()
            pltpu.make_async_copy(x2_hbm.at[pl.ds(base, _CHUNK)], b2, semX).wait()

        def compute_idx(xs, idx_ref):
            b0s, b1s, b2s = xs

            def grp(g, carry):
                s = pl.ds(g * 16, 16)
                xv, yv, zv = b0s[s], b1s[s], b2s[s]
                for li, l in enumerate(range(_NLOC, _LEVELS)):
                    res = jnp.float32(_RES[l])
                    pxi = (xv * res).astype(jnp.int32)
                    pyi = (yv * res).astype(jnp.int32)
                    pzi = (zv * res).astype(jnp.int32)
                    a = (pxi, pxi + 1)
                    b0 = pyi * jnp.int32(_P2)
                    c0 = pzi * jnp.int32(_P3)
                    bc = (b0 ^ c0, b0 ^ (c0 + jnp.int32(_P3)),
                          (b0 + jnp.int32(_P2)) ^ c0,
                          (b0 + jnp.int32(_P2)) ^ (c0 + jnp.int32(_P3)))
                    for corner in range(8):
                        h = (a[corner >> 2] ^ bc[corner & 3]) & jnp.int32(_MASK)
                        sl = pl.ds((li * 8 + corner) * _CHUNK + g * 16, 16)
                        idx_ref[sl] = h + jnp.int32(l * _T)
                return carry
            lax.fori_loop(0, _CHUNK // 16, grp, 0)

        def fire(ci, xs, semX, idx_ref, v_ref, sem):
            xdrain(ci, *xs, semX)
            compute_idx(xs, idx_ref)
            pltpu.async_copy(tpk_hbm.at[idx_ref], v_ref, sem)

        def drain_gather(idx_ref, v_ref, sem):
            pltpu.make_async_copy(tpk_hbm.at[idx_ref], v_ref, sem).wait()

        def accum_store(ci, it, xs, semX, v_ref, enc_v, semS):
            b0s, b1s, b2s = xs

            @pl.when(it >= 1)
            def _():
                pltpu.make_async_copy(
                    enc_v, enc_hbm.at[:, pl.ds(base0, _CHUNK)], semS).wait()

            def grp(g, carry):
                s = pl.ds(g * 16, 16)
                xv, yv, zv = b0s[s], b1s[s], b2s[s]
                for l in range(_NLOC):
                    res = jnp.float32(_RES[l])
                    posx = xv * res
                    posy = yv * res
                    posz = zv * res
                    pxi = posx.astype(jnp.int32)
                    pyi = posy.astype(jnp.int32)
                    pzi = posz.astype(jnp.int32)
                    fx = posx - pxi.astype(jnp.float32)
                    fy = posy - pyi.astype(jnp.float32)
                    fz = posz - pzi.astype(jnp.float32)
                    wx = (jnp.float32(1.0) - fx, fx)
                    wy = (jnp.float32(1.0) - fy, fy)
                    wz = (jnp.float32(1.0) - fz, fz)
                    wyz = (wy[0] * wz[0], wy[0] * wz[1],
                           wy[1] * wz[0], wy[1] * wz[1])
                    S = _S[l]
                    cell = (pxi * S + pyi) * S + pzi
                    acc0 = acc1 = None
                    for corner in range(8):
                        i3, j3, k3 = corner >> 2, (corner >> 1) & 1, corner & 1
                        doff = (i3 * S + j3) * S + k3 + _GOFF[l]
                        cvec = cell + jnp.int32(doff)
                        ui = plsc.bitcast(
                            plsc.load_gather(gridP, [cvec]), jnp.int32)
                        v0 = plsc.bitcast(ui & jnp.int32(-65536), jnp.float32)
                        v1 = plsc.bitcast(
                            lax.shift_left(ui, jnp.int32(16)), jnp.float32)
                        wc = wx[i3] * wyz[corner & 3]
                        if corner == 0:
                            acc0, acc1 = wc * v0, wc * v1
                        else:
                            acc0, acc1 = acc0 + wc * v0, acc1 + wc * v1
                    enc_v[2 * l, s] = acc0
                    enc_v[2 * l + 1, s] = acc1
                for li, l in enumerate(range(_NLOC, _LEVELS)):
                    res = jnp.float32(_RES[l])
                    posx = xv * res
                    posy = yv * res
                    posz = zv * res
                    fx = posx - posx.astype(jnp.int32).astype(jnp.float32)
                    fy = posy - posy.astype(jnp.int32).astype(jnp.float32)
                    fz = posz - posz.astype(jnp.int32).astype(jnp.float32)
                    wx = (jnp.float32(1.0) - fx, fx)
                    wy = (jnp.float32(1.0) - fy, fy)
                    wz = (jnp.float32(1.0) - fz, fz)
                    wyz = (wy[0] * wz[0], wy[0] * wz[1],
                           wy[1] * wz[0], wy[1] * wz[1])
                    acc0 = acc1 = None
                    for corner in range(8):
                        sv = pl.ds((li * 8 + corner) * _CHUNK + g * 16, 16)
                        ui = plsc.bitcast(v_ref[sv], jnp.int32)
                        v0 = plsc.bitcast(ui & jnp.int32(-65536), jnp.float32)
                        v1 = plsc.bitcast(
                            lax.shift_left(ui, jnp.int32(16)), jnp.float32)
                        wc = wx[corner >> 2] * wyz[corner & 3]
                        if corner == 0:
                            acc0, acc1 = wc * v0, wc * v1
                        else:
                            acc0, acc1 = acc0 + wc * v0, acc1 + wc * v1
                    enc_v[2 * l, s] = acc0
                    enc_v[2 * l + 1, s] = acc1
                return carry
            lax.fori_loop(0, _CHUNK // 16, grp, 0)
            base = base0 + ci * _CHUNK
            pltpu.async_copy(enc_v, enc_hbm.at[:, pl.ds(base, _CHUNK)], semS)

            @pl.when(ci + 2 < nchunk)
            def _():
                xload(ci + 2, b0s, b1s, b2s, semX)

        xsA = (xsA0, xsA1, xsA2)
        xsB = (xsB0, xsB1, xsB2)
        xload(0, *xsA, semXA)
        xload(1, *xsB, semXB)
        fire(0, xsA, semXA, idxA, valsA, semGA)

        def outer(it, carry):
            ci = 2 * it
            fire(ci + 1, xsB, semXB, idxB, valsB, semGB)
            drain_gather(idxA, valsA, semGA)
            accum_store(ci, it, xsA, semXA, valsA, encA, semSA)

            @pl.when(it < nchunk // 2 - 1)
            def _():
                fire(ci + 2, xsA, semXA, idxA, valsA, semGA)

            drain_gather(idxB, valsB, semGB)
            accum_store(ci + 1, it, xsB, semXB, valsB, encB, semSB)
            return carry

        lax.fori_loop(0, nchunk // 2, outer, 0)
        pltpu.make_async_copy(
            encA, enc_hbm.at[:, pl.ds(base0, _CHUNK)], semSA).wait()
        pltpu.make_async_copy(
            encB, enc_hbm.at[:, pl.ds(base0, _CHUNK)], semSB).wait()

    return enc_kernel(xc0, xc1, xc2, tpk, ghidx)


_BMLP = 4096


def _mlp_body(enc_ref, w1_ref, w2_ref, w3_ref, out_ref):
    e = enc_ref[...]
    h = jnp.maximum(
        jnp.dot(w1_ref[...], e, preferred_element_type=jnp.float32), 0.0)
    h = jnp.maximum(
        jnp.dot(w2_ref[...], h, preferred_element_type=jnp.float32), 0.0)
    out_ref[...] = jnp.dot(
        w3_ref[...], h, preferred_element_type=jnp.float32)[0]


def _mlp(enc, W1T, W2T, W3T):
    n = enc.shape[1]
    grid = (n // _BMLP,)
    return pl.pallas_call(
        _mlp_body,
        grid=grid,
        in_specs=[
            pl.BlockSpec((_LEVELS * _F, _BMLP), lambda i: (0, i)),
            pl.BlockSpec((64, _LEVELS * _F), lambda i: (0, 0)),
            pl.BlockSpec((64, 64), lambda i: (0, 0)),
            pl.BlockSpec((1, 64), lambda i: (0, 0)),
        ],
        out_specs=pl.BlockSpec((_BMLP,), lambda i: (i,)),
        out_shape=jax.ShapeDtypeStruct((n,), jnp.float32),
    )(enc, W1T, W2T, W3T)


_GHIDX = _grid_hash_indices()


def kernel(x, tables, W1, W2, W3):
    n = x.shape[0]
    xc0, xc1, xc2 = x[:, 0], x[:, 1], x[:, 2]
    b0 = lax.bitcast_convert_type(
        tables[:, :, 0].astype(jnp.bfloat16), jnp.uint16).astype(jnp.uint32)
    b1 = lax.bitcast_convert_type(
        tables[:, :, 1].astype(jnp.bfloat16), jnp.uint16).astype(jnp.uint32)
    tpk = lax.bitcast_convert_type(
        (b0 << 16) | b1, jnp.float32).reshape(-1)
    ghidx = jnp.asarray(_GHIDX)
    enc = _encode(xc0, xc1, xc2, tpk, ghidx, n)
    out = _mlp(enc, W1.T, W2.T, W3.T)
    return out.reshape(x.shape[:-1])
